# baseline (speedup 1.0000x reference)
import jax
import jax.numpy as jnp
from jax.experimental import pallas as pl


def _copy_body(x_ref, o_ref):
    o_ref[...] = x_ref[...]


def kernel(x, gate_w, expert_w, expert_b):
    b, n, d = x.shape
    x2 = x.reshape(b * n, d)
    rows = b * n
    block_rows = 2048
    out = pl.pallas_call(
        _copy_body,
        grid=(rows // block_rows,),
        in_specs=[pl.BlockSpec((block_rows, d), lambda i: (i, 0))],
        out_specs=pl.BlockSpec((block_rows, d), lambda i: (i, 0)),
        out_shape=jax.ShapeDtypeStruct((rows, d), x.dtype),
    )(x2)
    return out.reshape(b, n, d)
